# Initial kernel scaffold; baseline (speedup 1.0000x reference)
import jax
import jax.numpy as jnp
from jax.experimental import pallas as pl


def kernel(inputs, hidden_state, support_src, support_dst, support_vals, W_gate, b_gate, W_cand, b_cand, W_proj, b_proj):
    raise NotImplementedError("write your pallas kernel here")


# speedup vs baseline: 2.2879x; 2.2879x over previous
import functools

import jax
import jax.numpy as jnp
from jax import lax
from jax.experimental import pallas as pl
from jax.experimental.pallas import tpu as pltpu
from jax.experimental.pallas import tpu_sc as plsc

N = 10000
NP = 10240
E = 160000
U = 64
B = 4
F = 72
NSC = 2
NT = 16
CPS = 2 * F
LG = CPS // 16
K = 128
EPT = 10240
NCH = EPT // K
RPT = NP // NT
RCH = 128
NBLK = 2000


def _spmm_body(x_hbm, src_hbm, dst_hbm, val_hbm, out_hbm,
               acc, idxb, dstb, valb, rows, sem):
    cid = lax.axis_index("c")
    sid = lax.axis_index("s")
    wid = cid * NT + sid

    zero16 = jnp.zeros((16,), jnp.float32)

    def zrow(i, carry):
        for j in range(LG):
            rows[i, pl.ds(j * 16, 16)] = zero16
        return carry

    lax.fori_loop(0, RCH, zrow, 0)
    for kk in range(RPT // RCH):
        pltpu.sync_copy(rows, acc.at[pl.ds(sid * RPT + kk * RCH, RCH)])
    plsc.subcore_barrier()

    def chunk(ch, carry):
        pltpu.sync_copy(src_hbm.at[wid, ch], idxb)
        pltpu.sync_copy(dst_hbm.at[sid, ch], dstb)
        pltpu.sync_copy(val_hbm.at[sid, ch], valb)
        pltpu.async_copy(x_hbm.at[idxb], rows, sem).wait()

        def egroup(g, gcarry):
            v16 = valb[pl.ds(g * 16, 16)]
            for e16 in range(16):
                vv = lax.gather(
                    v16, jnp.full((16, 1), e16, jnp.int32),
                    lax.GatherDimensionNumbers(
                        offset_dims=(), collapsed_slice_dims=(0,),
                        start_index_map=(0,)),
                    (1,), mode=lax.GatherScatterMode.PROMISE_IN_BOUNDS)
                e = g * 16 + e16
                for j in range(LG):
                    sl = pl.ds(j * 16, 16)
                    rows[e, sl] = rows[e, sl] * vv
            return gcarry

        lax.fori_loop(0, K // 16, egroup, 0)
        pltpu.sync_copy(rows, acc.at[dstb], add=True)
        return carry

    lax.fori_loop(0, NCH, chunk, 0)
    plsc.subcore_barrier()

    for kk in range(RPT // RCH):
        r0 = sid * RPT + kk * RCH
        pltpu.sync_copy(acc.at[pl.ds(r0, RCH)], rows)
        pltpu.sync_copy(rows, out_hbm.at[pl.ds(cid * NP + r0, RCH)])


@functools.lru_cache(maxsize=None)
def _make_spmm():
    mesh = plsc.VectorSubcoreMesh(core_axis_name="c", subcore_axis_name="s",
                                  num_cores=NSC, num_subcores=NT)
    scratch = [
        pltpu.VMEM_SHARED((NP, CPS), jnp.float32),
        pltpu.VMEM((K,), jnp.int32),
        pltpu.VMEM((K,), jnp.int32),
        pltpu.VMEM((K,), jnp.float32),
        pltpu.VMEM((K, CPS), jnp.float32),
        pltpu.SemaphoreType.DMA,
    ]
    return pl.kernel(
        _spmm_body,
        out_type=jax.ShapeDtypeStruct((NSC * NP, CPS), jnp.float32),
        mesh=mesh,
        scratch_types=scratch,
        compiler_params=pltpu.CompilerParams(use_tc_tiling_on_sc=False),
    )


def _spmm(x, src2, dst3, val3):
    return _make_spmm()(x, src2, dst3, val3)


def _gate_body(x0_ref, x1_ref, x2_ref, wg_ref, bg_ref, it_ref, h_ref,
               u_ref, x0c_ref):
    pad = jnp.zeros((NBLK, F - 1 - U), jnp.float32)
    for c in range(NSC):
        for bl in range(2):
            b = 2 * c + bl
            sl = slice(bl * F, (bl + 1) * F)
            z = (jnp.dot(x0_ref[c][:, sl], wg_ref[0],
                         preferred_element_type=jnp.float32)
                 + jnp.dot(x1_ref[c][:, sl], wg_ref[1],
                           preferred_element_type=jnp.float32)
                 + jnp.dot(x2_ref[c][:, sl], wg_ref[2],
                           preferred_element_type=jnp.float32)
                 + bg_ref[...])
            v = jax.nn.sigmoid(z)
            r = v[:, :U]
            u_ref[b] = v[:, U:]
            x0c_ref[c, :, sl] = jnp.concatenate(
                [it_ref[:, b:b + 1], r * h_ref[b], pad], axis=1)


def _tc_gate(x0, x1, x2, wg, bg, it, h3):
    grid = (N // NBLK,)
    xspec = pl.BlockSpec((NSC, NBLK, CPS), lambda i: (0, i, 0))
    return pl.pallas_call(
        _gate_body,
        grid=grid,
        in_specs=[
            xspec, xspec, xspec,
            pl.BlockSpec((3, F, 2 * U), lambda i: (0, 0, 0)),
            pl.BlockSpec((1, 2 * U), lambda i: (0, 0)),
            pl.BlockSpec((NBLK, B), lambda i: (i, 0)),
            pl.BlockSpec((B, NBLK, U), lambda i: (0, i, 0)),
        ],
        out_specs=[
            pl.BlockSpec((B, NBLK, U), lambda i: (0, i, 0)),
            pl.BlockSpec((NSC, NBLK, CPS), lambda i: (0, i, 0)),
        ],
        out_shape=[
            jax.ShapeDtypeStruct((B, N, U), jnp.float32),
            jax.ShapeDtypeStruct((NSC, NP, CPS), jnp.float32),
        ],
    )(x0, x1, x2, wg, bg, it, h3)


def _cand_body(x0_ref, x1_ref, x2_ref, wc_ref, bc_ref, u_ref, h_ref,
               wp_ref, bp_ref, nh_ref, op_ref):
    for c in range(NSC):
        for bl in range(2):
            b = 2 * c + bl
            sl = slice(bl * F, (bl + 1) * F)
            z = (jnp.dot(x0_ref[c][:, sl], wc_ref[0],
                         preferred_element_type=jnp.float32)
                 + jnp.dot(x1_ref[c][:, sl], wc_ref[1],
                           preferred_element_type=jnp.float32)
                 + jnp.dot(x2_ref[c][:, sl], wc_ref[2],
                           preferred_element_type=jnp.float32)
                 + bc_ref[...])
            cc = jnp.tanh(z)
            uu = u_ref[b]
            nh = uu * h_ref[b] + (1.0 - uu) * cc
            nh_ref[b] = nh
            op_ref[:, b:b + 1] = (
                jnp.dot(nh, wp_ref[...], preferred_element_type=jnp.float32)
                + bp_ref[...])


def _tc_cand(x0, x1, x2, wc, bc, u, h3, wp, bp):
    grid = (N // NBLK,)
    xspec = pl.BlockSpec((NSC, NBLK, CPS), lambda i: (0, i, 0))
    uspec = pl.BlockSpec((B, NBLK, U), lambda i: (0, i, 0))
    return pl.pallas_call(
        _cand_body,
        grid=grid,
        in_specs=[
            xspec, xspec, xspec,
            pl.BlockSpec((3, F, U), lambda i: (0, 0, 0)),
            pl.BlockSpec((1, U), lambda i: (0, 0)),
            uspec, uspec,
            pl.BlockSpec((U, 1), lambda i: (0, 0)),
            pl.BlockSpec((1, 1), lambda i: (0, 0)),
        ],
        out_specs=[
            uspec,
            pl.BlockSpec((NBLK, B), lambda i: (i, 0)),
        ],
        out_shape=[
            jax.ShapeDtypeStruct((B, N, U), jnp.float32),
            jax.ShapeDtypeStruct((N, B), jnp.float32),
        ],
    )(x0, x1, x2, wc, bc, u, h3, wp, bp)


def _fold_cheb(w):
    return jnp.stack([w[0] - w[2], w[1], 2.0 * w[2]])


def kernel(inputs, hidden_state, support_src, support_dst, support_vals,
           W_gate, b_gate, W_cand, b_cand, W_proj, b_proj):
    h3 = hidden_state[0].reshape(B, N, U)
    xi = inputs.reshape(B, N, 1)
    pad = jnp.zeros((B, N, F - 1 - U), jnp.float32)
    xg = jnp.concatenate([xi, h3, pad], axis=2)
    x0g = xg.reshape(NSC, 2, N, F).transpose(0, 2, 1, 3).reshape(NSC, N, CPS)
    x0g = jnp.pad(x0g, ((0, 0), (0, NP - N), (0, 0))).reshape(NSC * NP, CPS)
    inputs_t = inputs.T

    npad = NT * EPT - E
    srcp = jnp.concatenate([support_src, jnp.zeros((npad,), jnp.int32)])
    dstp = jnp.concatenate([support_dst, jnp.zeros((npad,), jnp.int32)])
    valp = jnp.concatenate([support_vals, jnp.zeros((npad,), jnp.float32)])
    src2 = jnp.stack([srcp, srcp + NP]).reshape(NSC * NT, NCH, K)
    dst3 = dstp.reshape(NT, NCH, K)
    val3 = valp.reshape(NT, NCH, K)

    wg = _fold_cheb(jnp.pad(W_gate.reshape(F - 7, 3, 2 * U).transpose(1, 0, 2),
                            ((0, 0), (0, 7), (0, 0))))
    wc = _fold_cheb(jnp.pad(W_cand.reshape(F - 7, 3, U).transpose(1, 0, 2),
                            ((0, 0), (0, 7), (0, 0))))
    bg = b_gate.reshape(1, 2 * U)
    bc = b_cand.reshape(1, U)
    bp = b_proj.reshape(1, 1)

    x1g = _spmm(x0g, src2, dst3, val3)
    a2g = _spmm(x1g, src2, dst3, val3)
    u, x0c = _tc_gate(x0g.reshape(NSC, NP, CPS), x1g.reshape(NSC, NP, CPS),
                      a2g.reshape(NSC, NP, CPS), wg, bg, inputs_t, h3)
    x0c2 = x0c.reshape(NSC * NP, CPS)
    x1c = _spmm(x0c2, src2, dst3, val3)
    a2c = _spmm(x1c, src2, dst3, val3)
    newh, outp = _tc_cand(x0c, x1c.reshape(NSC, NP, CPS),
                          a2c.reshape(NSC, NP, CPS), wc, bc, u, h3,
                          W_proj, bp)
    output = outp.T.reshape(B, N)
    return output, jnp.stack([newh.reshape(B, N * U)], axis=0)


# speedup vs baseline: 2.5867x; 1.1306x over previous
import functools

import jax
import jax.numpy as jnp
from jax import lax
from jax.experimental import pallas as pl
from jax.experimental.pallas import tpu as pltpu
from jax.experimental.pallas import tpu_sc as plsc

N = 10000
NP = 10240
E = 160000
U = 64
B = 4
F = 72
NSC = 2
NT = 16
CPS = 2 * F
LG = CPS // 16
K = 256
EPT = 10240
NCH = EPT // K
RPT = NP // NT
RCH = 256
NBLK = 2000


def _spmm_body(x_hbm, src_hbm, dst_hbm, val_hbm, out_hbm,
               acc, idxb, dstb, valb, rows, sem):
    cid = lax.axis_index("c")
    sid = lax.axis_index("s")
    wid = cid * NT + sid

    zero16 = jnp.zeros((16,), jnp.float32)

    def zrow(i, carry):
        for j in range(LG):
            rows[i, pl.ds(j * 16, 16)] = zero16
        return carry

    lax.fori_loop(0, RCH, zrow, 0)
    for kk in range(RPT // RCH):
        pltpu.sync_copy(rows, acc.at[pl.ds(sid * RPT + kk * RCH, RCH)])
    plsc.subcore_barrier()

    def chunk(ch, carry):
        pltpu.sync_copy(src_hbm.at[wid, ch], idxb)
        pltpu.sync_copy(dst_hbm.at[sid, ch], dstb)
        pltpu.sync_copy(val_hbm.at[sid, ch], valb)
        pltpu.async_copy(x_hbm.at[idxb], rows, sem).wait()

        def egroup(g, gcarry):
            v16 = valb[pl.ds(g * 16, 16)]
            for e16 in range(16):
                vv = lax.gather(
                    v16, jnp.full((16, 1), e16, jnp.int32),
                    lax.GatherDimensionNumbers(
                        offset_dims=(), collapsed_slice_dims=(0,),
                        start_index_map=(0,)),
                    (1,), mode=lax.GatherScatterMode.PROMISE_IN_BOUNDS)
                e = g * 16 + e16
                for j in range(LG):
                    sl = pl.ds(j * 16, 16)
                    rows[e, sl] = rows[e, sl] * vv
            return gcarry

        lax.fori_loop(0, K // 16, egroup, 0)
        pltpu.sync_copy(rows, acc.at[dstb], add=True)
        return carry

    lax.fori_loop(0, NCH, chunk, 0)
    plsc.subcore_barrier()

    for kk in range(RPT // RCH):
        r0 = sid * RPT + kk * RCH
        pltpu.sync_copy(acc.at[pl.ds(r0, RCH)], rows)
        pltpu.sync_copy(rows, out_hbm.at[pl.ds(cid * NP + r0, RCH)])


@functools.lru_cache(maxsize=None)
def _make_spmm():
    mesh = plsc.VectorSubcoreMesh(core_axis_name="c", subcore_axis_name="s",
                                  num_cores=NSC, num_subcores=NT)
    scratch = [
        pltpu.VMEM_SHARED((NP, CPS), jnp.float32),
        pltpu.VMEM((K,), jnp.int32),
        pltpu.VMEM((K,), jnp.int32),
        pltpu.VMEM((K,), jnp.float32),
        pltpu.VMEM((K, CPS), jnp.float32),
        pltpu.SemaphoreType.DMA,
    ]
    return pl.kernel(
        _spmm_body,
        out_type=jax.ShapeDtypeStruct((NSC * NP, CPS), jnp.float32),
        mesh=mesh,
        scratch_types=scratch,
        compiler_params=pltpu.CompilerParams(use_tc_tiling_on_sc=False),
    )


def _spmm(x, src2, dst3, val3):
    return _make_spmm()(x, src2, dst3, val3)


def _gate_body(x0_ref, x1_ref, x2_ref, wg_ref, bg_ref, it_ref, h_ref,
               u_ref, x0c_ref):
    pad = jnp.zeros((NBLK, F - 1 - U), jnp.float32)
    for c in range(NSC):
        for bl in range(2):
            b = 2 * c + bl
            sl = slice(bl * F, (bl + 1) * F)
            z = (jnp.dot(x0_ref[c][:, sl], wg_ref[0],
                         preferred_element_type=jnp.float32)
                 + jnp.dot(x1_ref[c][:, sl], wg_ref[1],
                           preferred_element_type=jnp.float32)
                 + jnp.dot(x2_ref[c][:, sl], wg_ref[2],
                           preferred_element_type=jnp.float32)
                 + bg_ref[...])
            v = jax.nn.sigmoid(z)
            r = v[:, :U]
            u_ref[b] = v[:, U:]
            x0c_ref[c, :, sl] = jnp.concatenate(
                [it_ref[:, b:b + 1], r * h_ref[b], pad], axis=1)


def _tc_gate(x0, x1, x2, wg, bg, it, h3):
    grid = (N // NBLK,)
    xspec = pl.BlockSpec((NSC, NBLK, CPS), lambda i: (0, i, 0))
    return pl.pallas_call(
        _gate_body,
        grid=grid,
        in_specs=[
            xspec, xspec, xspec,
            pl.BlockSpec((3, F, 2 * U), lambda i: (0, 0, 0)),
            pl.BlockSpec((1, 2 * U), lambda i: (0, 0)),
            pl.BlockSpec((NBLK, B), lambda i: (i, 0)),
            pl.BlockSpec((B, NBLK, U), lambda i: (0, i, 0)),
        ],
        out_specs=[
            pl.BlockSpec((B, NBLK, U), lambda i: (0, i, 0)),
            pl.BlockSpec((NSC, NBLK, CPS), lambda i: (0, i, 0)),
        ],
        out_shape=[
            jax.ShapeDtypeStruct((B, N, U), jnp.float32),
            jax.ShapeDtypeStruct((NSC, NP, CPS), jnp.float32),
        ],
    )(x0, x1, x2, wg, bg, it, h3)


def _cand_body(x0_ref, x1_ref, x2_ref, wc_ref, bc_ref, u_ref, h_ref,
               wp_ref, bp_ref, nh_ref, op_ref):
    for c in range(NSC):
        for bl in range(2):
            b = 2 * c + bl
            sl = slice(bl * F, (bl + 1) * F)
            z = (jnp.dot(x0_ref[c][:, sl], wc_ref[0],
                         preferred_element_type=jnp.float32)
                 + jnp.dot(x1_ref[c][:, sl], wc_ref[1],
                           preferred_element_type=jnp.float32)
                 + jnp.dot(x2_ref[c][:, sl], wc_ref[2],
                           preferred_element_type=jnp.float32)
                 + bc_ref[...])
            cc = jnp.tanh(z)
            uu = u_ref[b]
            nh = uu * h_ref[b] + (1.0 - uu) * cc
            nh_ref[b] = nh
            op_ref[:, b:b + 1] = (
                jnp.dot(nh, wp_ref[...], preferred_element_type=jnp.float32)
                + bp_ref[...])


def _tc_cand(x0, x1, x2, wc, bc, u, h3, wp, bp):
    grid = (N // NBLK,)
    xspec = pl.BlockSpec((NSC, NBLK, CPS), lambda i: (0, i, 0))
    uspec = pl.BlockSpec((B, NBLK, U), lambda i: (0, i, 0))
    return pl.pallas_call(
        _cand_body,
        grid=grid,
        in_specs=[
            xspec, xspec, xspec,
            pl.BlockSpec((3, F, U), lambda i: (0, 0, 0)),
            pl.BlockSpec((1, U), lambda i: (0, 0)),
            uspec, uspec,
            pl.BlockSpec((U, 1), lambda i: (0, 0)),
            pl.BlockSpec((1, 1), lambda i: (0, 0)),
        ],
        out_specs=[
            uspec,
            pl.BlockSpec((NBLK, B), lambda i: (i, 0)),
        ],
        out_shape=[
            jax.ShapeDtypeStruct((B, N, U), jnp.float32),
            jax.ShapeDtypeStruct((N, B), jnp.float32),
        ],
    )(x0, x1, x2, wc, bc, u, h3, wp, bp)


def _fold_cheb(w):
    return jnp.stack([w[0] - w[2], w[1], 2.0 * w[2]])


def kernel(inputs, hidden_state, support_src, support_dst, support_vals,
           W_gate, b_gate, W_cand, b_cand, W_proj, b_proj):
    h3 = hidden_state[0].reshape(B, N, U)
    xi = inputs.reshape(B, N, 1)
    pad = jnp.zeros((B, N, F - 1 - U), jnp.float32)
    xg = jnp.concatenate([xi, h3, pad], axis=2)
    x0g = xg.reshape(NSC, 2, N, F).transpose(0, 2, 1, 3).reshape(NSC, N, CPS)
    x0g = jnp.pad(x0g, ((0, 0), (0, NP - N), (0, 0))).reshape(NSC * NP, CPS)
    inputs_t = inputs.T

    npad = NT * EPT - E
    srcp = jnp.concatenate([support_src, jnp.zeros((npad,), jnp.int32)])
    dstp = jnp.concatenate([support_dst, jnp.zeros((npad,), jnp.int32)])
    valp = jnp.concatenate([support_vals, jnp.zeros((npad,), jnp.float32)])
    src2 = jnp.stack([srcp, srcp + NP]).reshape(NSC * NT, NCH, K)
    dst3 = dstp.reshape(NT, NCH, K)
    val3 = valp.reshape(NT, NCH, K)

    wg = _fold_cheb(jnp.pad(W_gate.reshape(F - 7, 3, 2 * U).transpose(1, 0, 2),
                            ((0, 0), (0, 7), (0, 0))))
    wc = _fold_cheb(jnp.pad(W_cand.reshape(F - 7, 3, U).transpose(1, 0, 2),
                            ((0, 0), (0, 7), (0, 0))))
    bg = b_gate.reshape(1, 2 * U)
    bc = b_cand.reshape(1, U)
    bp = b_proj.reshape(1, 1)

    x1g = _spmm(x0g, src2, dst3, val3)
    a2g = _spmm(x1g, src2, dst3, val3)
    u, x0c = _tc_gate(x0g.reshape(NSC, NP, CPS), x1g.reshape(NSC, NP, CPS),
                      a2g.reshape(NSC, NP, CPS), wg, bg, inputs_t, h3)
    x0c2 = x0c.reshape(NSC * NP, CPS)
    x1c = _spmm(x0c2, src2, dst3, val3)
    a2c = _spmm(x1c, src2, dst3, val3)
    newh, outp = _tc_cand(x0c, x1c.reshape(NSC, NP, CPS),
                          a2c.reshape(NSC, NP, CPS), wc, bc, u, h3,
                          W_proj, bp)
    output = outp.T.reshape(B, N)
    return output, jnp.stack([newh.reshape(B, N * U)], axis=0)


# speedup vs baseline: 3.1697x; 1.2254x over previous
import functools

import jax
import jax.numpy as jnp
from jax import lax
from jax.experimental import pallas as pl
from jax.experimental.pallas import tpu as pltpu
from jax.experimental.pallas import tpu_sc as plsc

N = 10000
NP = 10240
E = 160000
U = 64
B = 4
F = 72
NSC = 2
NT = 16
CPS = 2 * F
LG = CPS // 16
K = 128
EPT = 10240
NCH = EPT // K
RPT = NP // NT
RCH = 128
NBLK = 2000


def _spmm_body(x_hbm, edge_hbm, out_hbm, acc,
               ebuf0, ebuf1, rows0, rows1,
               esem0, esem1, gsem0, gsem1, ssem0, ssem1):
    cid = lax.axis_index("c")
    sid = lax.axis_index("s")
    wid = cid * NT + sid
    ebufs = (ebuf0, ebuf1)
    rowss = (rows0, rows1)
    esems = (esem0, esem1)
    gsems = (gsem0, gsem1)
    ssems = (ssem0, ssem1)

    def ecopy_start(ch, t):
        pltpu.async_copy(edge_hbm.at[wid, ch], ebufs[t], esems[t])

    def ecopy_wait(ch, t):
        pltpu.make_async_copy(edge_hbm.at[wid, ch], ebufs[t],
                              esems[t]).wait()

    def gather_start(t):
        pltpu.async_copy(x_hbm.at[ebufs[t].at[0]], rowss[t], gsems[t])

    def gather_wait(t):
        pltpu.make_async_copy(x_hbm.at[ebufs[t].at[0]], rowss[t],
                              gsems[t]).wait()

    def scatter_start(t):
        pltpu.async_copy(rowss[t], acc.at[ebufs[t].at[1]], ssems[t],
                         add=True)

    def scatter_wait(t):
        pltpu.make_async_copy(rowss[t], acc.at[ebufs[t].at[1]],
                              ssems[t]).wait()

    def scale(t):
        rb = rowss[t]
        eb = ebufs[t]

        @plsc.parallel_loop(0, K // 16, unroll=2)
        def _(g):
            v16 = lax.bitcast_convert_type(eb[2, pl.ds(g * 16, 16)], jnp.float32)
            for e16 in range(16):
                vv = lax.gather(
                    v16, jnp.full((16, 1), e16, jnp.int32),
                    lax.GatherDimensionNumbers(
                        offset_dims=(), collapsed_slice_dims=(0,),
                        start_index_map=(0,)),
                    (1,), mode=lax.GatherScatterMode.PROMISE_IN_BOUNDS)
                e = g * 16 + e16
                for j in range(LG):
                    sl = pl.ds(j * 16, 16)
                    rb[e, sl] = rb[e, sl] * vv

    zero16 = jnp.zeros((16,), jnp.float32)

    def zrow(i, carry):
        for j in range(LG):
            rows0[i, pl.ds(j * 16, 16)] = zero16
        return carry

    lax.fori_loop(0, RCH, zrow, 0)
    for kk in range(RPT // RCH):
        pltpu.sync_copy(rows0, acc.at[pl.ds(sid * RPT + kk * RCH, RCH)])

    ecopy_start(0, 0)
    ecopy_wait(0, 0)
    gather_start(0)
    plsc.subcore_barrier()

    def chunk_iter(ch, t):
        nt_ = 1 - t

        @pl.when(ch >= 1)
        def _():
            scatter_wait(nt_)

        @pl.when(ch + 1 < NCH)
        def _():
            ecopy_start(ch + 1, nt_)
            ecopy_wait(ch + 1, nt_)
            gather_start(nt_)

        gather_wait(t)
        scale(t)
        scatter_start(t)

    def pair(p, carry):
        chunk_iter(2 * p, 0)
        chunk_iter(2 * p + 1, 1)
        return carry

    lax.fori_loop(0, NCH // 2, pair, 0)
    scatter_wait((NCH - 1) % 2)
    plsc.subcore_barrier()

    nwo = RPT // RCH
    for kk in range(nwo):
        t = kk % 2
        r0 = sid * RPT + kk * RCH

        def owait(kq, tq):
            rq = sid * RPT + kq * RCH
            pltpu.make_async_copy(
                rowss[tq], out_hbm.at[pl.ds(cid * NP + rq, RCH)],
                gsems[tq]).wait()

        if kk >= 2:
            owait(kk - 2, t)
        pltpu.sync_copy(acc.at[pl.ds(r0, RCH)], rowss[t])
        pltpu.async_copy(rowss[t], out_hbm.at[pl.ds(cid * NP + r0, RCH)],
                         gsems[t])
    for kk in range(max(nwo - 2, 0), nwo):
        rq = sid * RPT + kk * RCH
        pltpu.make_async_copy(
            rowss[kk % 2], out_hbm.at[pl.ds(cid * NP + rq, RCH)],
            gsems[kk % 2]).wait()


@functools.lru_cache(maxsize=None)
def _make_spmm():
    mesh = plsc.VectorSubcoreMesh(core_axis_name="c", subcore_axis_name="s",
                                  num_cores=NSC, num_subcores=NT)
    scratch = [
        pltpu.VMEM_SHARED((NP, CPS), jnp.float32),
        pltpu.VMEM((3, K), jnp.int32),
        pltpu.VMEM((3, K), jnp.int32),
        pltpu.VMEM((K, CPS), jnp.float32),
        pltpu.VMEM((K, CPS), jnp.float32),
        pltpu.SemaphoreType.DMA,
        pltpu.SemaphoreType.DMA,
        pltpu.SemaphoreType.DMA,
        pltpu.SemaphoreType.DMA,
        pltpu.SemaphoreType.DMA,
        pltpu.SemaphoreType.DMA,
    ]
    return pl.kernel(
        _spmm_body,
        out_type=jax.ShapeDtypeStruct((NSC * NP, CPS), jnp.float32),
        mesh=mesh,
        scratch_types=scratch,
        compiler_params=pltpu.CompilerParams(use_tc_tiling_on_sc=False),
    )


def _spmm(x, edges):
    return _make_spmm()(x, edges)


def _gate_body(x0_ref, x1_ref, x2_ref, wg_ref, bg_ref, it_ref, h_ref,
               u_ref, x0c_ref):
    pad = jnp.zeros((NBLK, F - 1 - U), jnp.float32)
    for c in range(NSC):
        for bl in range(2):
            b = 2 * c + bl
            sl = slice(bl * F, (bl + 1) * F)
            z = (jnp.dot(x0_ref[c][:, sl], wg_ref[0],
                         preferred_element_type=jnp.float32)
                 + jnp.dot(x1_ref[c][:, sl], wg_ref[1],
                           preferred_element_type=jnp.float32)
                 + jnp.dot(x2_ref[c][:, sl], wg_ref[2],
                           preferred_element_type=jnp.float32)
                 + bg_ref[...])
            v = jax.nn.sigmoid(z)
            r = v[:, :U]
            u_ref[b] = v[:, U:]
            x0c_ref[c, :, sl] = jnp.concatenate(
                [it_ref[:, b:b + 1], r * h_ref[b], pad], axis=1)


def _tc_gate(x0, x1, x2, wg, bg, it, h3):
    grid = (N // NBLK,)
    xspec = pl.BlockSpec((NSC, NBLK, CPS), lambda i: (0, i, 0))
    return pl.pallas_call(
        _gate_body,
        grid=grid,
        in_specs=[
            xspec, xspec, xspec,
            pl.BlockSpec((3, F, 2 * U), lambda i: (0, 0, 0)),
            pl.BlockSpec((1, 2 * U), lambda i: (0, 0)),
            pl.BlockSpec((NBLK, B), lambda i: (i, 0)),
            pl.BlockSpec((B, NBLK, U), lambda i: (0, i, 0)),
        ],
        out_specs=[
            pl.BlockSpec((B, NBLK, U), lambda i: (0, i, 0)),
            pl.BlockSpec((NSC, NBLK, CPS), lambda i: (0, i, 0)),
        ],
        out_shape=[
            jax.ShapeDtypeStruct((B, N, U), jnp.float32),
            jax.ShapeDtypeStruct((NSC, NP, CPS), jnp.float32),
        ],
    )(x0, x1, x2, wg, bg, it, h3)


def _cand_body(x0_ref, x1_ref, x2_ref, wc_ref, bc_ref, u_ref, h_ref,
               wp_ref, bp_ref, nh_ref, op_ref):
    for c in range(NSC):
        for bl in range(2):
            b = 2 * c + bl
            sl = slice(bl * F, (bl + 1) * F)
            z = (jnp.dot(x0_ref[c][:, sl], wc_ref[0],
                         preferred_element_type=jnp.float32)
                 + jnp.dot(x1_ref[c][:, sl], wc_ref[1],
                           preferred_element_type=jnp.float32)
                 + jnp.dot(x2_ref[c][:, sl], wc_ref[2],
                           preferred_element_type=jnp.float32)
                 + bc_ref[...])
            cc = jnp.tanh(z)
            uu = u_ref[b]
            nh = uu * h_ref[b] + (1.0 - uu) * cc
            nh_ref[b] = nh
            op_ref[:, b:b + 1] = (
                jnp.dot(nh, wp_ref[...], preferred_element_type=jnp.float32)
                + bp_ref[...])


def _tc_cand(x0, x1, x2, wc, bc, u, h3, wp, bp):
    grid = (N // NBLK,)
    xspec = pl.BlockSpec((NSC, NBLK, CPS), lambda i: (0, i, 0))
    uspec = pl.BlockSpec((B, NBLK, U), lambda i: (0, i, 0))
    return pl.pallas_call(
        _cand_body,
        grid=grid,
        in_specs=[
            xspec, xspec, xspec,
            pl.BlockSpec((3, F, U), lambda i: (0, 0, 0)),
            pl.BlockSpec((1, U), lambda i: (0, 0)),
            uspec, uspec,
            pl.BlockSpec((U, 1), lambda i: (0, 0)),
            pl.BlockSpec((1, 1), lambda i: (0, 0)),
        ],
        out_specs=[
            uspec,
            pl.BlockSpec((NBLK, B), lambda i: (i, 0)),
        ],
        out_shape=[
            jax.ShapeDtypeStruct((B, N, U), jnp.float32),
            jax.ShapeDtypeStruct((N, B), jnp.float32),
        ],
    )(x0, x1, x2, wc, bc, u, h3, wp, bp)


def _fold_cheb(w):
    return jnp.stack([w[0] - w[2], w[1], 2.0 * w[2]])


def kernel(inputs, hidden_state, support_src, support_dst, support_vals,
           W_gate, b_gate, W_cand, b_cand, W_proj, b_proj):
    h3 = hidden_state[0].reshape(B, N, U)
    xi = inputs.reshape(B, N, 1)
    pad = jnp.zeros((B, N, F - 1 - U), jnp.float32)
    xg = jnp.concatenate([xi, h3, pad], axis=2)
    x0g = xg.reshape(NSC, 2, N, F).transpose(0, 2, 1, 3).reshape(NSC, N, CPS)
    x0g = jnp.pad(x0g, ((0, 0), (0, NP - N), (0, 0))).reshape(NSC * NP, CPS)
    inputs_t = inputs.T

    npad = NT * EPT - E
    srcp = jnp.concatenate([support_src, jnp.zeros((npad,), jnp.int32)])
    dstp = jnp.concatenate([support_dst, jnp.zeros((npad,), jnp.int32)])
    valp = jnp.concatenate([support_vals, jnp.zeros((npad,), jnp.float32)])
    vbits = lax.bitcast_convert_type(valp, jnp.int32).reshape(1, NT, NCH, 1, K)
    dst4 = dstp.reshape(1, NT, NCH, 1, K)
    edges = jnp.concatenate([
        jnp.stack([srcp, srcp + NP]).reshape(NSC, NT, NCH, 1, K),
        jnp.concatenate([dst4, dst4], axis=0),
        jnp.concatenate([vbits, vbits], axis=0),
    ], axis=3).reshape(NSC * NT, NCH, 3, K)

    wg = _fold_cheb(jnp.pad(W_gate.reshape(F - 7, 3, 2 * U).transpose(1, 0, 2),
                            ((0, 0), (0, 7), (0, 0))))
    wc = _fold_cheb(jnp.pad(W_cand.reshape(F - 7, 3, U).transpose(1, 0, 2),
                            ((0, 0), (0, 7), (0, 0))))
    bg = b_gate.reshape(1, 2 * U)
    bc = b_cand.reshape(1, U)
    bp = b_proj.reshape(1, 1)

    x1g = _spmm(x0g, edges)
    a2g = _spmm(x1g, edges)
    u, x0c = _tc_gate(x0g.reshape(NSC, NP, CPS), x1g.reshape(NSC, NP, CPS),
                      a2g.reshape(NSC, NP, CPS), wg, bg, inputs_t, h3)
    x0c2 = x0c.reshape(NSC * NP, CPS)
    x1c = _spmm(x0c2, edges)
    a2c = _spmm(x1c, edges)
    newh, outp = _tc_cand(x0c, x1c.reshape(NSC, NP, CPS),
                          a2c.reshape(NSC, NP, CPS), wc, bc, u, h3,
                          W_proj, bp)
    output = outp.T.reshape(B, N)
    return output, jnp.stack([newh.reshape(B, N * U)], axis=0)


# speedup vs baseline: 4.8391x; 1.5267x over previous
import functools

import jax
import jax.numpy as jnp
from jax import lax
from jax.experimental import pallas as pl
from jax.experimental.pallas import tpu as pltpu
from jax.experimental.pallas import tpu_sc as plsc

N = 10000
NP = 10240
E = 160000
U = 64
B = 4
FB = 80
NSC = 2
NT = 16
CPB = 2 * FB
LGB = CPB // 32
K = 160
EPT = 10240
NCH = EPT // K
RPT = NP // NT
NWO = RPT // K
NBUF = 4
NBLK = 2000


def _spmm_body(x_hbm, edge_hbm, vs_hbm, out_hbm, acc, *bufs):
    ebufs = bufs[0:NBUF]
    vbufs = bufs[NBUF:2 * NBUF]
    rowss = bufs[2 * NBUF:3 * NBUF]
    esems = bufs[3 * NBUF:4 * NBUF]
    vsems = bufs[4 * NBUF:5 * NBUF]
    gsems = bufs[5 * NBUF:6 * NBUF]
    ssems = bufs[6 * NBUF:7 * NBUF]
    cid = lax.axis_index("c")
    sid = lax.axis_index("s")
    wid = cid * NT + sid

    def ecopy_start(ch, s):
        pltpu.async_copy(edge_hbm.at[wid, ch], ebufs[s], esems[s])
        pltpu.async_copy(vs_hbm.at[sid, ch], vbufs[s], vsems[s])

    def ecopy_wait(ch, s):
        pltpu.make_async_copy(edge_hbm.at[wid, ch], ebufs[s],
                              esems[s]).wait()
        pltpu.make_async_copy(vs_hbm.at[sid, ch], vbufs[s],
                              vsems[s]).wait()

    def gather_start(s):
        pltpu.async_copy(x_hbm.at[ebufs[s].at[0]], rowss[s], gsems[s])

    def gather_wait(s):
        pltpu.make_async_copy(x_hbm.at[ebufs[s].at[0]], rowss[s],
                              gsems[s]).wait()

    def scatter_start(s):
        pltpu.async_copy(rowss[s], acc.at[ebufs[s].at[1]], ssems[s],
                         add=True)

    def scatter_wait(s):
        pltpu.make_async_copy(rowss[s], acc.at[ebufs[s].at[1]],
                              ssems[s]).wait()

    def scale(s):
        rb = rowss[s]
        vb = vbufs[s]

        @plsc.parallel_loop(0, K, unroll=4)
        def _(e):
            vvb = vb[e, pl.ds(0, 32)]
            for j in range(LGB):
                sl = pl.ds(j * 32, 32)
                rb[e, sl] = rb[e, sl] * vvb

    zero32 = jnp.zeros((32,), jnp.bfloat16)

    def zrow(i, carry):
        for j in range(LGB):
            rowss[0][i, pl.ds(j * 32, 32)] = zero32
        return carry

    lax.fori_loop(0, K, zrow, 0)
    for kk in range(NWO):
        pltpu.sync_copy(rowss[0], acc.at[pl.ds(sid * RPT + kk * K, K)])

    ecopy_start(0, 0)
    ecopy_start(1, 1)
    ecopy_wait(0, 0)
    gather_start(0)
    plsc.subcore_barrier()

    def chunk_iter(ch, t):
        @pl.when(ch >= 2)
        def _():
            scatter_wait((t + 2) % NBUF)

        @pl.when(ch + 2 < NCH)
        def _():
            ecopy_start(ch + 2, (t + 2) % NBUF)

        @pl.when(ch + 1 < NCH)
        def _():
            ecopy_wait(ch + 1, (t + 1) % NBUF)
            gather_start((t + 1) % NBUF)

        gather_wait(t)
        scale(t)
        scatter_start(t)

    def quad(p, carry):
        for t in range(NBUF):
            chunk_iter(NBUF * p + t, t)
        return carry

    lax.fori_loop(0, NCH // NBUF, quad, 0)
    scatter_wait((NCH - 2) % NBUF)
    scatter_wait((NCH - 1) % NBUF)
    plsc.subcore_barrier()

    for kk in range(NWO):
        r0 = sid * RPT + kk * K
        pltpu.sync_copy(acc.at[pl.ds(r0, K)], rowss[kk])
        pltpu.async_copy(rowss[kk], out_hbm.at[pl.ds(cid * NP + r0, K)],
                         gsems[kk])
    for kk in range(NWO):
        r0 = sid * RPT + kk * K
        pltpu.make_async_copy(
            rowss[kk], out_hbm.at[pl.ds(cid * NP + r0, K)],
            gsems[kk]).wait()


@functools.lru_cache(maxsize=None)
def _make_spmm():
    mesh = plsc.VectorSubcoreMesh(core_axis_name="c", subcore_axis_name="s",
                                  num_cores=NSC, num_subcores=NT)
    scratch = (
        [pltpu.VMEM_SHARED((NP, CPB), jnp.bfloat16)]
        + [pltpu.VMEM((2, K), jnp.int32) for _ in range(NBUF)]
        + [pltpu.VMEM((K, 32), jnp.bfloat16) for _ in range(NBUF)]
        + [pltpu.VMEM((K, CPB), jnp.bfloat16) for _ in range(NBUF)]
        + [pltpu.SemaphoreType.DMA for _ in range(4 * NBUF)]
    )
    return pl.kernel(
        _spmm_body,
        out_type=jax.ShapeDtypeStruct((NSC * NP, CPB), jnp.bfloat16),
        mesh=mesh,
        scratch_types=scratch,
        compiler_params=pltpu.CompilerParams(use_tc_tiling_on_sc=False),
    )


def _spmm(x, edges, vsplat):
    return _make_spmm()(x, edges, vsplat)


def _prep_body(it_ref, h_ref, x0_ref):
    pad = jnp.zeros((NBLK, FB - 1 - U), jnp.float32)
    for c in range(NSC):
        for bl in range(2):
            b = 2 * c + bl
            sl = slice(bl * FB, (bl + 1) * FB)
            x0_ref[c, :, sl] = jnp.concatenate(
                [it_ref[:, b:b + 1], h_ref[b], pad],
                axis=1).astype(jnp.bfloat16)


def _tc_prep(it, h3):
    grid = (N // NBLK,)
    return pl.pallas_call(
        _prep_body,
        grid=grid,
        in_specs=[
            pl.BlockSpec((NBLK, B), lambda i: (i, 0)),
            pl.BlockSpec((B, NBLK, U), lambda i: (0, i, 0)),
        ],
        out_specs=pl.BlockSpec((NSC, NBLK, CPB), lambda i: (0, i, 0)),
        out_shape=jax.ShapeDtypeStruct((NSC, NP, CPB), jnp.bfloat16),
    )(it, h3)


def _gate_body(x0_ref, x1_ref, x2_ref, wg_ref, bg_ref, it_ref, h_ref,
               u_ref, x0c_ref):
    pad = jnp.zeros((NBLK, FB - 1 - U), jnp.float32)
    for c in range(NSC):
        for bl in range(2):
            b = 2 * c + bl
            sl = slice(bl * FB, (bl + 1) * FB)
            z = (jnp.dot(x0_ref[c][:, sl].astype(jnp.float32), wg_ref[0],
                         preferred_element_type=jnp.float32)
                 + jnp.dot(x1_ref[c][:, sl].astype(jnp.float32), wg_ref[1],
                           preferred_element_type=jnp.float32)
                 + jnp.dot(x2_ref[c][:, sl].astype(jnp.float32), wg_ref[2],
                           preferred_element_type=jnp.float32)
                 + bg_ref[...])
            v = jax.nn.sigmoid(z)
            r = v[:, :U]
            u_ref[b] = v[:, U:]
            x0c_ref[c, :, sl] = jnp.concatenate(
                [it_ref[:, b:b + 1], r * h_ref[b], pad],
                axis=1).astype(jnp.bfloat16)


def _tc_gate(x0, x1, x2, wg, bg, it, h3):
    grid = (N // NBLK,)
    xspec = pl.BlockSpec((NSC, NBLK, CPB), lambda i: (0, i, 0))
    return pl.pallas_call(
        _gate_body,
        grid=grid,
        in_specs=[
            xspec, xspec, xspec,
            pl.BlockSpec((3, FB, 2 * U), lambda i: (0, 0, 0)),
            pl.BlockSpec((1, 2 * U), lambda i: (0, 0)),
            pl.BlockSpec((NBLK, B), lambda i: (i, 0)),
            pl.BlockSpec((B, NBLK, U), lambda i: (0, i, 0)),
        ],
        out_specs=[
            pl.BlockSpec((B, NBLK, U), lambda i: (0, i, 0)),
            pl.BlockSpec((NSC, NBLK, CPB), lambda i: (0, i, 0)),
        ],
        out_shape=[
            jax.ShapeDtypeStruct((B, N, U), jnp.float32),
            jax.ShapeDtypeStruct((NSC, NP, CPB), jnp.bfloat16),
        ],
    )(x0, x1, x2, wg, bg, it, h3)


def _cand_body(x0_ref, x1_ref, x2_ref, wc_ref, bc_ref, u_ref, h_ref,
               wp_ref, bp_ref, nh_ref, op_ref):
    for c in range(NSC):
        for bl in range(2):
            b = 2 * c + bl
            sl = slice(bl * FB, (bl + 1) * FB)
            z = (jnp.dot(x0_ref[c][:, sl].astype(jnp.float32), wc_ref[0],
                         preferred_element_type=jnp.float32)
                 + jnp.dot(x1_ref[c][:, sl].astype(jnp.float32), wc_ref[1],
                           preferred_element_type=jnp.float32)
                 + jnp.dot(x2_ref[c][:, sl].astype(jnp.float32), wc_ref[2],
                           preferred_element_type=jnp.float32)
                 + bc_ref[...])
            cc = jnp.tanh(z)
            uu = u_ref[b]
            nh = uu * h_ref[b] + (1.0 - uu) * cc
            nh_ref[b] = nh
            op_ref[:, b:b + 1] = (
                jnp.dot(nh, wp_ref[...], preferred_element_type=jnp.float32)
                + bp_ref[...])


def _tc_cand(x0, x1, x2, wc, bc, u, h3, wp, bp):
    grid = (N // NBLK,)
    xspec = pl.BlockSpec((NSC, NBLK, CPB), lambda i: (0, i, 0))
    uspec = pl.BlockSpec((B, NBLK, U), lambda i: (0, i, 0))
    return pl.pallas_call(
        _cand_body,
        grid=grid,
        in_specs=[
            xspec, xspec, xspec,
            pl.BlockSpec((3, FB, U), lambda i: (0, 0, 0)),
            pl.BlockSpec((1, U), lambda i: (0, 0)),
            uspec, uspec,
            pl.BlockSpec((U, 1), lambda i: (0, 0)),
            pl.BlockSpec((1, 1), lambda i: (0, 0)),
        ],
        out_specs=[
            uspec,
            pl.BlockSpec((NBLK, B), lambda i: (i, 0)),
        ],
        out_shape=[
            jax.ShapeDtypeStruct((B, N, U), jnp.float32),
            jax.ShapeDtypeStruct((N, B), jnp.float32),
        ],
    )(x0, x1, x2, wc, bc, u, h3, wp, bp)


def _fold_cheb(w):
    return jnp.stack([w[0] - w[2], w[1], 2.0 * w[2]])


def kernel(inputs, hidden_state, support_src, support_dst, support_vals,
           W_gate, b_gate, W_cand, b_cand, W_proj, b_proj):
    h3 = hidden_state[0].reshape(B, N, U)
    inputs_t = inputs.T

    npad = NT * EPT - E
    srcp = jnp.concatenate([support_src, jnp.zeros((npad,), jnp.int32)])
    dstp = jnp.concatenate([support_dst, jnp.zeros((npad,), jnp.int32)])
    valp = jnp.concatenate([support_vals, jnp.zeros((npad,), jnp.float32)])
    dst4 = dstp.reshape(1, NT, NCH, 1, K)
    edges = jnp.concatenate([
        jnp.stack([srcp, srcp + NP]).reshape(NSC, NT, NCH, 1, K),
        jnp.concatenate([dst4, dst4], axis=0),
    ], axis=3).reshape(NSC * NT, NCH, 2, K)
    vsplat = jnp.broadcast_to(valp.astype(jnp.bfloat16)[:, None],
                              (NT * EPT, 32)).reshape(NT, NCH, K, 32)

    wg = _fold_cheb(jnp.pad(W_gate.reshape(65, 3, 2 * U).transpose(1, 0, 2),
                            ((0, 0), (0, FB - 65), (0, 0))))
    wc = _fold_cheb(jnp.pad(W_cand.reshape(65, 3, U).transpose(1, 0, 2),
                            ((0, 0), (0, FB - 65), (0, 0))))
    bg = b_gate.reshape(1, 2 * U)
    bc = b_cand.reshape(1, U)
    bp = b_proj.reshape(1, 1)

    x0g = _tc_prep(inputs_t, h3)
    x0g2 = x0g.reshape(NSC * NP, CPB)
    x1g = _spmm(x0g2, edges, vsplat)
    a2g = _spmm(x1g, edges, vsplat)
    u, x0c = _tc_gate(x0g, x1g.reshape(NSC, NP, CPB),
                      a2g.reshape(NSC, NP, CPB), wg, bg, inputs_t, h3)
    x0c2 = x0c.reshape(NSC * NP, CPB)
    x1c = _spmm(x0c2, edges, vsplat)
    a2c = _spmm(x1c, edges, vsplat)
    newh, outp = _tc_cand(x0c, x1c.reshape(NSC, NP, CPB),
                          a2c.reshape(NSC, NP, CPB), wc, bc, u, h3,
                          W_proj, bp)
    output = outp.T.reshape(B, N)
    return output, jnp.stack([newh.reshape(B, N * U)], axis=0)
